# Initial kernel scaffold; baseline (speedup 1.0000x reference)
import functools

import jax
import jax.numpy as jnp
from jax import lax
from jax.experimental import pallas as pl
from jax.experimental.pallas import tpu as pltpu
from jax.experimental.pallas import tpu_sc as plsc

N = 10000
E = 320000
H = 128
C = 40
G = 64

NC = 2
NS = 16
NW = NC * NS
EPW = E // NW
K = 80
NCHUNK = EPW // K
RPT = N // NS

_MESH = plsc.VectorSubcoreMesh(
    core_axis_name="c", subcore_axis_name="s", num_cores=NC, num_subcores=NS
)

_HIGH = lax.Precision.HIGHEST


def _dot(a, b):
    return lax.dot_general(
        a, b, (((1,), (0,)), ((), ())),
        precision=_HIGH, preferred_element_type=jnp.float32,
    )


@functools.partial(
    pl.kernel,
    out_type=jax.ShapeDtypeStruct((NC * N, 8), jnp.float32),
    mesh=_MESH,
    scratch_types=[
        pltpu.VMEM_SHARED((N, 8), jnp.float32),
        pltpu.VMEM((K,), jnp.int32),
        pltpu.VMEM((K, 8), jnp.float32),
    ],
)
def _deg_kernel(dst_hbm, z8_hbm, ones8_hbm, out_hbm, acc, idx_v, ones_v):
    c = lax.axis_index("c")
    s = lax.axis_index("s")
    pltpu.sync_copy(z8_hbm, acc.at[pl.ds(s * RPT, RPT)])
    pltpu.sync_copy(ones8_hbm, ones_v)
    plsc.subcore_barrier()
    base = (c * NS + s) * EPW

    def body(j, carry):
        pltpu.sync_copy(dst_hbm.at[pl.ds(base + j * K, K)], idx_v)
        pltpu.sync_copy(ones_v, acc.at[idx_v], add=True)
        return carry

    lax.fori_loop(0, NCHUNK, body, 0)
    plsc.subcore_barrier()
    pltpu.sync_copy(
        acc.at[pl.ds(s * RPT, RPT)],
        out_hbm.at[pl.ds(c * N + s * RPT, RPT)],
    )


@functools.partial(
    pl.kernel,
    out_type=jax.ShapeDtypeStruct((NC * N, H), jnp.float32),
    mesh=_MESH,
    scratch_types=[
        pltpu.VMEM_SHARED((N, H), jnp.float32),
        pltpu.VMEM((K,), jnp.int32),
        pltpu.VMEM((K,), jnp.int32),
        pltpu.VMEM((K, H), jnp.float32),
        pltpu.SemaphoreType.DMA,
    ],
)
def _edge_kernel(y_hbm, src_hbm, dst_hbm, z128_hbm, out_hbm,
                 acc, si_v, di_v, rows_v, sem):
    c = lax.axis_index("c")
    s = lax.axis_index("s")
    pltpu.sync_copy(z128_hbm, acc.at[pl.ds(s * RPT, RPT)])
    plsc.subcore_barrier()
    base = (c * NS + s) * EPW

    def body(j, carry):
        off = base + j * K
        pltpu.sync_copy(src_hbm.at[pl.ds(off, K)], si_v)
        pltpu.sync_copy(dst_hbm.at[pl.ds(off, K)], di_v)
        pltpu.async_copy(y_hbm.at[si_v], rows_v, sem).wait()
        pltpu.sync_copy(rows_v, acc.at[di_v], add=True)
        return carry

    lax.fori_loop(0, NCHUNK, body, 0)
    plsc.subcore_barrier()
    pltpu.sync_copy(
        acc.at[pl.ds(s * RPT, RPT)],
        out_hbm.at[pl.ds(c * N + s * RPT, RPT)],
    )


def _tc_first_body(x_ref, w_ref, deg_ref, ty_ref, dinv_ref):
    d = deg_ref[0:N, 0:1] + deg_ref[N:2 * N, 0:1]
    dinv = lax.rsqrt(d + 1.0)
    dinv_ref[...] = dinv
    ty_ref[...] = _dot(x_ref[...], w_ref[...]) * dinv


_tc_first = pl.pallas_call(
    _tc_first_body,
    out_shape=[
        jax.ShapeDtypeStruct((N, H), jnp.float32),
        jax.ShapeDtypeStruct((N, 1), jnp.float32),
    ],
)


def _tc_mid_body(u_ref, typ_ref, dinv_ref, b_ref, w_ref, out_ref):
    dinv = dinv_ref[...]
    z = (u_ref[0:N] + u_ref[N:2 * N] + typ_ref[...]) * dinv + b_ref[...]
    out_ref[...] = _dot(jnp.tanh(z), w_ref[...]) * dinv


_tc_mid = pl.pallas_call(
    _tc_mid_body,
    out_shape=jax.ShapeDtypeStruct((N, H), jnp.float32),
)


def _tc_final_body(u_ref, typ_ref, dinv_ref, b_ref, batch_ref, wout_ref,
                   bout_ref, out_ref):
    h3 = (u_ref[0:N] + u_ref[N:2 * N] + typ_ref[...]) * dinv_ref[...] + b_ref[...]
    oh = (lax.broadcasted_iota(jnp.int32, (G, N), 0) == batch_ref[...])
    oh = oh.astype(jnp.float32)
    sums = _dot(oh, h3)
    cnt = jnp.sum(oh, axis=1, keepdims=True)
    pooled = sums / jnp.maximum(cnt, 1.0)
    logits = _dot(pooled, wout_ref[...]) + bout_ref[...]
    m = jnp.max(logits, axis=1, keepdims=True)
    lse = jnp.log(jnp.sum(jnp.exp(logits - m), axis=1, keepdims=True))
    out_ref[...] = logits - m - lse


_tc_final = pl.pallas_call(
    _tc_final_body,
    out_shape=jax.ShapeDtypeStruct((G, C), jnp.float32),
)


def kernel(x, edge_index, batch, W0, b0, W1, b1, W2, b2, Wout, bout):
    src = edge_index[0]
    dst = edge_index[1]
    z8 = jnp.zeros((RPT, 8), jnp.float32)
    z128 = jnp.zeros((RPT, H), jnp.float32)
    ones8 = jnp.ones((K, 8), jnp.float32)
    batch_row = batch.reshape(1, N)

    degparts = _deg_kernel(dst, z8, ones8)
    ty0, dinv = _tc_first(x, W0, degparts)
    u0 = _edge_kernel(ty0, src, dst, z128)
    ty1 = _tc_mid(u0, ty0, dinv, b0, W1)
    u1 = _edge_kernel(ty1, src, dst, z128)
    ty2 = _tc_mid(u1, ty1, dinv, b1, W2)
    u2 = _edge_kernel(ty2, src, dst, z128)
    return _tc_final(u2, ty2, dinv, b2, batch_row, Wout, bout)


# speedup vs baseline: 9.2527x; 9.2527x over previous
import functools

import jax
import jax.numpy as jnp
from jax import lax
from jax.experimental import pallas as pl
from jax.experimental.pallas import tpu as pltpu
from jax.experimental.pallas import tpu_sc as plsc

N = 10000
E = 320000
H = 128
C = 40
G = 64

NC = 2
NS = 16
NW = NC * NS
EPW = E // NW
K = 80
NCHUNK = EPW // K
RPT = 624
REM = N - NS * RPT

_MESH = plsc.VectorSubcoreMesh(
    core_axis_name="c", subcore_axis_name="s", num_cores=NC, num_subcores=NS
)

_HIGH = lax.Precision.HIGHEST


def _dot(a, b):
    return lax.dot_general(
        a, b, (((1,), (0,)), ((), ())),
        precision=_HIGH, preferred_element_type=jnp.float32,
    )


@functools.partial(
    pl.kernel,
    out_type=jax.ShapeDtypeStruct((NC * N, H), jnp.float32),
    mesh=_MESH,
    scratch_types=[
        pltpu.VMEM_SHARED((N, H), jnp.float32),
        pltpu.VMEM((K,), jnp.int32),
        pltpu.VMEM((K,), jnp.int32),
        pltpu.VMEM((K, H), jnp.float32),
        pltpu.SemaphoreType.DMA,
    ],
)
def _edge_kernel(y_hbm, src_hbm, dst_hbm, z128_hbm, out_hbm,
                 acc, si_v, di_v, rows_v, sem):
    c = lax.axis_index("c")
    s = lax.axis_index("s")
    pltpu.sync_copy(z128_hbm, acc.at[pl.ds(s * RPT, RPT)])
    @pl.when(s == 0)
    def _():
        pltpu.sync_copy(z128_hbm.at[pl.ds(0, REM)], acc.at[pl.ds(NS * RPT, REM)])
    plsc.subcore_barrier()
    base = (c * NS + s) * EPW

    def body(j, carry):
        off = base + j * K
        pltpu.sync_copy(src_hbm.at[pl.ds(off, K)], si_v)
        pltpu.sync_copy(dst_hbm.at[pl.ds(off, K)], di_v)
        pltpu.async_copy(y_hbm.at[si_v], rows_v, sem).wait()
        pltpu.sync_copy(rows_v, acc.at[di_v], add=True)
        return carry

    lax.fori_loop(0, NCHUNK, body, 0)
    plsc.subcore_barrier()
    pltpu.sync_copy(
        acc.at[pl.ds(s * RPT, RPT)],
        out_hbm.at[pl.ds(c * N + s * RPT, RPT)],
    )
    @pl.when(s == 0)
    def _():
        pltpu.sync_copy(
            acc.at[pl.ds(NS * RPT, REM)],
            out_hbm.at[pl.ds(c * N + NS * RPT, REM)],
        )


def _tc_first_body(x_ref, w_ref, deg_ref, ty_ref, dinv_ref):
    d = deg_ref[0:N, 0:1] + deg_ref[N:2 * N, 0:1]
    dinv = lax.rsqrt(d + 1.0)
    dinv_ref[...] = dinv
    ty_ref[...] = _dot(x_ref[...], w_ref[...]) * dinv


_tc_first = pl.pallas_call(
    _tc_first_body,
    out_shape=[
        jax.ShapeDtypeStruct((N, H), jnp.float32),
        jax.ShapeDtypeStruct((N, 1), jnp.float32),
    ],
)


def _tc_mid_body(u_ref, typ_ref, dinv_ref, b_ref, w_ref, out_ref):
    dinv = dinv_ref[...]
    z = (u_ref[0:N] + u_ref[N:2 * N] + typ_ref[...]) * dinv + b_ref[...]
    out_ref[...] = _dot(jnp.tanh(z), w_ref[...]) * dinv


_tc_mid = pl.pallas_call(
    _tc_mid_body,
    out_shape=jax.ShapeDtypeStruct((N, H), jnp.float32),
)


def _tc_final_body(u_ref, typ_ref, dinv_ref, b_ref, batch_ref, wout_ref,
                   bout_ref, out_ref):
    h3 = (u_ref[0:N] + u_ref[N:2 * N] + typ_ref[...]) * dinv_ref[...] + b_ref[...]
    oh = (lax.broadcasted_iota(jnp.int32, (G, N), 0) == batch_ref[...])
    oh = oh.astype(jnp.float32)
    sums = _dot(oh, h3)
    cnt = jnp.sum(oh, axis=1, keepdims=True)
    pooled = sums / jnp.maximum(cnt, 1.0)
    logits = _dot(pooled, wout_ref[...]) + bout_ref[...]
    m = jnp.max(logits, axis=1, keepdims=True)
    lse = jnp.log(jnp.sum(jnp.exp(logits - m), axis=1, keepdims=True))
    out_ref[...] = logits - m - lse


_tc_final = pl.pallas_call(
    _tc_final_body,
    out_shape=jax.ShapeDtypeStruct((G, C), jnp.float32),
)


def kernel(x, edge_index, batch, W0, b0, W1, b1, W2, b2, Wout, bout):
    src = edge_index[0]
    dst = edge_index[1]
    z128 = jnp.zeros((RPT, H), jnp.float32)
    ones_n = jnp.ones((N, H), jnp.float32)
    batch_row = batch.reshape(1, N)

    degparts = _edge_kernel(ones_n, dst, dst, z128)
    ty0, dinv = _tc_first(x, W0, degparts)
    u0 = _edge_kernel(ty0, src, dst, z128)
    ty1 = _tc_mid(u0, ty0, dinv, b0, W1)
    u1 = _edge_kernel(ty1, src, dst, z128)
    ty2 = _tc_mid(u1, ty1, dinv, b1, W2)
    u2 = _edge_kernel(ty2, src, dst, z128)
    return _tc_final(u2, ty2, dinv, b2, batch_row, Wout, bout)


# speedup vs baseline: 22.3650x; 2.4171x over previous
import functools

import jax
import jax.numpy as jnp
from jax import lax
from jax.experimental import pallas as pl
from jax.experimental.pallas import tpu as pltpu
from jax.experimental.pallas import tpu_sc as plsc

N = 10000
E = 320000
H = 128
C = 40
G = 64

NC = 2
NS = 16
NW = NC * NS
EPW = E // NW
K = 50
NBUF = 5
NGRP = EPW // (K * NBUF)
RPT = 624
REM = N - NS * RPT

_MESH = plsc.VectorSubcoreMesh(
    core_axis_name="c", subcore_axis_name="s", num_cores=NC, num_subcores=NS
)

_HIGH = lax.Precision.HIGHEST


def _dot(a, b):
    return lax.dot_general(
        a, b, (((1,), (0,)), ((), ())),
        precision=_HIGH, preferred_element_type=jnp.float32,
    )


@functools.partial(
    pl.kernel,
    out_type=jax.ShapeDtypeStruct((NC * N, H), jnp.float32),
    mesh=_MESH,
    scratch_types=(
        [
            pltpu.VMEM_SHARED((N, H), jnp.float32),
            pltpu.VMEM((2, NBUF, K), jnp.int32),
            pltpu.VMEM((2, NBUF, K), jnp.int32),
        ]
        + [pltpu.VMEM((K, H), jnp.float32)] * NBUF
        + [pltpu.SemaphoreType.DMA] * (2 + 2 * NBUF)
    ),
)
def _edge_kernel(y_hbm, src4_hbm, dst4_hbm, z128_hbm, out_hbm,
                 acc, sig, dig, *bufs):
    rows = bufs[:NBUF]
    isem_s = bufs[NBUF]
    isem_d = bufs[NBUF + 1]
    gsem = bufs[NBUF + 2:NBUF + 2 + NBUF]
    ssem = bufs[NBUF + 2 + NBUF:]
    c = lax.axis_index("c")
    s = lax.axis_index("s")
    wid = c * NS + s

    pltpu.sync_copy(z128_hbm, acc.at[pl.ds(s * RPT, RPT)])
    @pl.when(s == 0)
    def _():
        pltpu.sync_copy(z128_hbm.at[pl.ds(0, REM)], acc.at[pl.ds(NS * RPT, REM)])
    pltpu.sync_copy(src4_hbm.at[wid, 0], sig.at[0])
    pltpu.sync_copy(dst4_hbm.at[wid, 0], dig.at[0])
    plsc.subcore_barrier()

    def grp(i, carry):
        d = lax.rem(i, 2)
        @pl.when(i > 0)
        def _():
            pltpu.make_async_copy(src4_hbm.at[wid, i], sig.at[d], isem_s).wait()
            pltpu.make_async_copy(dst4_hbm.at[wid, i], dig.at[d], isem_d).wait()
        for b in range(NBUF):
            @pl.when(i > 0)
            def _():
                pltpu.make_async_copy(
                    rows[b], acc.at[dig.at[d, b]], ssem[b]
                ).wait()
            pltpu.async_copy(y_hbm.at[sig.at[d, b]], rows[b], gsem[b])
        @pl.when(i < NGRP - 1)
        def _():
            pltpu.async_copy(src4_hbm.at[wid, i + 1], sig.at[1 - d], isem_s)
            pltpu.async_copy(dst4_hbm.at[wid, i + 1], dig.at[1 - d], isem_d)
        for b in range(NBUF):
            pltpu.make_async_copy(y_hbm.at[sig.at[d, b]], rows[b], gsem[b]).wait()
            pltpu.async_copy(rows[b], acc.at[dig.at[d, b]], ssem[b], add=True)
        return carry

    lax.fori_loop(0, NGRP, grp, 0)
    for b in range(NBUF):
        pltpu.make_async_copy(
            rows[b], acc.at[dig.at[(NGRP - 1) % 2, b]], ssem[b]
        ).wait()
    plsc.subcore_barrier()
    pltpu.sync_copy(
        acc.at[pl.ds(s * RPT, RPT)],
        out_hbm.at[pl.ds(c * N + s * RPT, RPT)],
    )
    @pl.when(s == 0)
    def _():
        pltpu.sync_copy(
            acc.at[pl.ds(NS * RPT, REM)],
            out_hbm.at[pl.ds(c * N + NS * RPT, REM)],
        )


def _tc_first_body(x_ref, w_ref, deg_ref, ty_ref, dinv_ref):
    d = deg_ref[0:N, 0:1] + deg_ref[N:2 * N, 0:1]
    dinv = lax.rsqrt(d + 1.0)
    dinv_ref[...] = dinv
    ty_ref[...] = _dot(x_ref[...], w_ref[...]) * dinv


_tc_first = pl.pallas_call(
    _tc_first_body,
    out_shape=[
        jax.ShapeDtypeStruct((N, H), jnp.float32),
        jax.ShapeDtypeStruct((N, 1), jnp.float32),
    ],
)


def _tc_mid_body(u_ref, typ_ref, dinv_ref, b_ref, w_ref, out_ref):
    dinv = dinv_ref[...]
    z = (u_ref[0:N] + u_ref[N:2 * N] + typ_ref[...]) * dinv + b_ref[...]
    out_ref[...] = _dot(jnp.tanh(z), w_ref[...]) * dinv


_tc_mid = pl.pallas_call(
    _tc_mid_body,
    out_shape=jax.ShapeDtypeStruct((N, H), jnp.float32),
)


def _tc_final_body(u_ref, typ_ref, dinv_ref, b_ref, batch_ref, wout_ref,
                   bout_ref, out_ref):
    h3 = (u_ref[0:N] + u_ref[N:2 * N] + typ_ref[...]) * dinv_ref[...] + b_ref[...]
    oh = (lax.broadcasted_iota(jnp.int32, (G, N), 0) == batch_ref[...])
    oh = oh.astype(jnp.float32)
    sums = _dot(oh, h3)
    cnt = jnp.sum(oh, axis=1, keepdims=True)
    pooled = sums / jnp.maximum(cnt, 1.0)
    logits = _dot(pooled, wout_ref[...]) + bout_ref[...]
    m = jnp.max(logits, axis=1, keepdims=True)
    lse = jnp.log(jnp.sum(jnp.exp(logits - m), axis=1, keepdims=True))
    out_ref[...] = logits - m - lse


_tc_final = pl.pallas_call(
    _tc_final_body,
    out_shape=jax.ShapeDtypeStruct((G, C), jnp.float32),
)


def kernel(x, edge_index, batch, W0, b0, W1, b1, W2, b2, Wout, bout):
    src4 = edge_index[0].reshape(NW, NGRP, NBUF, K)
    dst4 = edge_index[1].reshape(NW, NGRP, NBUF, K)
    z128 = jnp.zeros((RPT, H), jnp.float32)
    ones_n = jnp.ones((N, H), jnp.float32)
    batch_row = batch.reshape(1, N)

    degparts = _edge_kernel(ones_n, dst4, dst4, z128)
    ty0, dinv = _tc_first(x, W0, degparts)
    u0 = _edge_kernel(ty0, src4, dst4, z128)
    ty1 = _tc_mid(u0, ty0, dinv, b0, W1)
    u1 = _edge_kernel(ty1, src4, dst4, z128)
    ty2 = _tc_mid(u1, ty1, dinv, b1, W2)
    u2 = _edge_kernel(ty2, src4, dst4, z128)
    return _tc_final(u2, ty2, dinv, b2, batch_row, Wout, bout)


# speedup vs baseline: 24.3958x; 1.0908x over previous
import functools

import jax
import jax.numpy as jnp
from jax import lax
from jax.experimental import pallas as pl
from jax.experimental.pallas import tpu as pltpu
from jax.experimental.pallas import tpu_sc as plsc

N = 10000
E = 320000
H = 128
C = 40
G = 64

NC = 2
NS = 16
NW = NC * NS
EPW = E // NW
K = 50
NBUF = 5
NGRP = EPW // (K * NBUF)
RPT = 624
REM = N - NS * RPT

_MESH = plsc.VectorSubcoreMesh(
    core_axis_name="c", subcore_axis_name="s", num_cores=NC, num_subcores=NS
)

_HIGH = lax.Precision.HIGHEST


def _dot(a, b):
    return lax.dot_general(
        a, b, (((1,), (0,)), ((), ())),
        precision=_HIGH, preferred_element_type=jnp.float32,
    )


@functools.partial(
    pl.kernel,
    out_type=jax.ShapeDtypeStruct((NC * N, H), jnp.float32),
    mesh=_MESH,
    scratch_types=(
        [
            pltpu.VMEM_SHARED((N, H), jnp.float32),
            pltpu.VMEM((2, NBUF, K), jnp.int32),
            pltpu.VMEM((2, NBUF, K), jnp.int32),
        ]
        + [pltpu.VMEM((K, H), jnp.float32)] * NBUF
        + [pltpu.SemaphoreType.DMA] * (2 + 2 * NBUF)
    ),
)
def _edge_kernel(y_hbm, src4_hbm, dst4_hbm, z128_hbm, out_hbm,
                 acc, sig, dig, *bufs):
    rows = bufs[:NBUF]
    isem_s = bufs[NBUF]
    isem_d = bufs[NBUF + 1]
    gsem = bufs[NBUF + 2:NBUF + 2 + NBUF]
    ssem = bufs[NBUF + 2 + NBUF:]
    c = lax.axis_index("c")
    s = lax.axis_index("s")
    wid = c * NS + s

    pltpu.sync_copy(z128_hbm, acc.at[pl.ds(s * RPT, RPT)])
    @pl.when(s == 0)
    def _():
        pltpu.sync_copy(z128_hbm.at[pl.ds(0, REM)], acc.at[pl.ds(NS * RPT, REM)])
    pltpu.sync_copy(src4_hbm.at[wid, 0], sig.at[0])
    pltpu.sync_copy(dst4_hbm.at[wid, 0], dig.at[0])
    plsc.subcore_barrier()

    def grp(i, carry):
        d = lax.rem(i, 2)
        @pl.when(i > 0)
        def _():
            pltpu.make_async_copy(src4_hbm.at[wid, i], sig.at[d], isem_s).wait()
            pltpu.make_async_copy(dst4_hbm.at[wid, i], dig.at[d], isem_d).wait()
        for b in range(NBUF):
            @pl.when(i > 0)
            def _():
                pltpu.make_async_copy(
                    rows[b], acc.at[dig.at[d, b]], ssem[b]
                ).wait()
            pltpu.async_copy(y_hbm.at[sig.at[d, b]], rows[b], gsem[b])
        @pl.when(i < NGRP - 1)
        def _():
            pltpu.async_copy(src4_hbm.at[wid, i + 1], sig.at[1 - d], isem_s)
            pltpu.async_copy(dst4_hbm.at[wid, i + 1], dig.at[1 - d], isem_d)
        for b in range(NBUF):
            pltpu.make_async_copy(y_hbm.at[sig.at[d, b]], rows[b], gsem[b]).wait()
            pltpu.async_copy(rows[b], acc.at[dig.at[d, b]], ssem[b], add=True)
        return carry

    lax.fori_loop(0, NGRP, grp, 0)
    for b in range(NBUF):
        pltpu.make_async_copy(
            rows[b], acc.at[dig.at[(NGRP - 1) % 2, b]], ssem[b]
        ).wait()
    plsc.subcore_barrier()
    pltpu.sync_copy(
        acc.at[pl.ds(s * RPT, RPT)],
        out_hbm.at[pl.ds(c * N + s * RPT, RPT)],
    )
    @pl.when(s == 0)
    def _():
        pltpu.sync_copy(
            acc.at[pl.ds(NS * RPT, REM)],
            out_hbm.at[pl.ds(c * N + NS * RPT, REM)],
        )


@functools.partial(
    pl.kernel,
    out_type=jax.ShapeDtypeStruct((NC * N, H), jnp.float32),
    mesh=_MESH,
    scratch_types=(
        [
            pltpu.VMEM_SHARED((N, H), jnp.float32),
            pltpu.VMEM((2, NBUF, K), jnp.int32),
            pltpu.VMEM((K, H), jnp.float32),
        ]
        + [pltpu.SemaphoreType.DMA] * (1 + NBUF)
    ),
)
def _deg_kernel(dst4_hbm, z128_hbm, ones_hbm, out_hbm, acc, dig, ob, *sems):
    isem_d = sems[0]
    ssem = sems[1:]
    c = lax.axis_index("c")
    s = lax.axis_index("s")
    wid = c * NS + s

    pltpu.sync_copy(z128_hbm, acc.at[pl.ds(s * RPT, RPT)])
    @pl.when(s == 0)
    def _():
        pltpu.sync_copy(z128_hbm.at[pl.ds(0, REM)], acc.at[pl.ds(NS * RPT, REM)])
    pltpu.sync_copy(ones_hbm, ob)
    pltpu.sync_copy(dst4_hbm.at[wid, 0], dig.at[0])
    plsc.subcore_barrier()

    def grp(i, carry):
        d = lax.rem(i, 2)
        @pl.when(i > 0)
        def _():
            pltpu.make_async_copy(dst4_hbm.at[wid, i], dig.at[d], isem_d).wait()
        for b in range(NBUF):
            @pl.when(i > 0)
            def _():
                pltpu.make_async_copy(ob, acc.at[dig.at[d, b]], ssem[b]).wait()
        @pl.when(i < NGRP - 1)
        def _():
            pltpu.async_copy(dst4_hbm.at[wid, i + 1], dig.at[1 - d], isem_d)
        for b in range(NBUF):
            pltpu.async_copy(ob, acc.at[dig.at[d, b]], ssem[b], add=True)
        return carry

    lax.fori_loop(0, NGRP, grp, 0)
    for b in range(NBUF):
        pltpu.make_async_copy(
            ob, acc.at[dig.at[(NGRP - 1) % 2, b]], ssem[b]
        ).wait()
    plsc.subcore_barrier()
    pltpu.sync_copy(
        acc.at[pl.ds(s * RPT, RPT)],
        out_hbm.at[pl.ds(c * N + s * RPT, RPT)],
    )
    @pl.when(s == 0)
    def _():
        pltpu.sync_copy(
            acc.at[pl.ds(NS * RPT, REM)],
            out_hbm.at[pl.ds(c * N + NS * RPT, REM)],
        )


def _tc_first_body(x_ref, w_ref, deg_ref, ty_ref, dinv_ref):
    d = deg_ref[0:N, 0:1] + deg_ref[N:2 * N, 0:1]
    dinv = lax.rsqrt(d + 1.0)
    dinv_ref[...] = dinv
    ty_ref[...] = _dot(x_ref[...], w_ref[...]) * dinv


_tc_first = pl.pallas_call(
    _tc_first_body,
    out_shape=[
        jax.ShapeDtypeStruct((N, H), jnp.float32),
        jax.ShapeDtypeStruct((N, 1), jnp.float32),
    ],
)


def _tc_mid_body(u_ref, typ_ref, dinv_ref, b_ref, w_ref, out_ref):
    dinv = dinv_ref[...]
    z = (u_ref[0:N] + u_ref[N:2 * N] + typ_ref[...]) * dinv + b_ref[...]
    out_ref[...] = _dot(jnp.tanh(z), w_ref[...]) * dinv


_tc_mid = pl.pallas_call(
    _tc_mid_body,
    out_shape=jax.ShapeDtypeStruct((N, H), jnp.float32),
)


def _tc_final_body(u_ref, typ_ref, dinv_ref, b_ref, batch_ref, wout_ref,
                   bout_ref, out_ref):
    h3 = (u_ref[0:N] + u_ref[N:2 * N] + typ_ref[...]) * dinv_ref[...] + b_ref[...]
    oh = (lax.broadcasted_iota(jnp.int32, (G, N), 0) == batch_ref[...])
    oh = oh.astype(jnp.float32)
    sums = _dot(oh, h3)
    cnt = jnp.sum(oh, axis=1, keepdims=True)
    pooled = sums / jnp.maximum(cnt, 1.0)
    logits = _dot(pooled, wout_ref[...]) + bout_ref[...]
    m = jnp.max(logits, axis=1, keepdims=True)
    lse = jnp.log(jnp.sum(jnp.exp(logits - m), axis=1, keepdims=True))
    out_ref[...] = logits - m - lse


_tc_final = pl.pallas_call(
    _tc_final_body,
    out_shape=jax.ShapeDtypeStruct((G, C), jnp.float32),
)


def kernel(x, edge_index, batch, W0, b0, W1, b1, W2, b2, Wout, bout):
    src4 = edge_index[0].reshape(NW, NGRP, NBUF, K)
    dst4 = edge_index[1].reshape(NW, NGRP, NBUF, K)
    z128 = jnp.zeros((RPT, H), jnp.float32)
    ones_k = jnp.ones((K, H), jnp.float32)
    batch_row = batch.reshape(1, N)

    degparts = _deg_kernel(dst4, z128, ones_k)
    ty0, dinv = _tc_first(x, W0, degparts)
    u0 = _edge_kernel(ty0, src4, dst4, z128)
    ty1 = _tc_mid(u0, ty0, dinv, b0, W1)
    u1 = _edge_kernel(ty1, src4, dst4, z128)
    ty2 = _tc_mid(u1, ty1, dinv, b1, W2)
    u2 = _edge_kernel(ty2, src4, dst4, z128)
    return _tc_final(u2, ty2, dinv, b2, batch_row, Wout, bout)
